# baseline (speedup 1.0000x reference)
import functools

import jax
import jax.numpy as jnp
from jax import lax
from jax.experimental import pallas as pl
from jax.experimental.pallas import tpu as pltpu
from jax.experimental.pallas import tpu_sc as plsc

BATCH = 16384
EMBED = 64
HIDDEN = 128

_NC, _NS = 2, 16
_NW = _NC * _NS
_B_PER_W = BATCH // _NW
_CHUNK = 128
_NCHUNK = _B_PER_W // _CHUNK
_IDX_COLS = 128


def _gather_body(uid_hbm, bid_hbm, user_emb, book_emb, ue_out, be_out,
                 idx_u, idx_b, rows_u, rows_b, sem_u, sem_b):
    wid = lax.axis_index("s") * _NC + lax.axis_index("c")
    base = wid * _B_PER_W
    row0 = wid * _NCHUNK
    pltpu.sync_copy(uid_hbm.at[pl.ds(row0, _NCHUNK)], idx_u)
    pltpu.sync_copy(bid_hbm.at[pl.ds(row0, _NCHUNK)], idx_b)
    cps = []
    for j in range(_NCHUNK):
        dst = pl.ds(j * _CHUNK, _CHUNK)
        cps.append(pltpu.async_copy(user_emb.at[idx_u.at[j]], rows_u.at[dst], sem_u))
        cps.append(pltpu.async_copy(book_emb.at[idx_b.at[j]], rows_b.at[dst], sem_b))
    for cp in cps:
        cp.wait()
    pltpu.sync_copy(rows_u, ue_out.at[pl.ds(base, _B_PER_W)])
    pltpu.sync_copy(rows_b, be_out.at[pl.ds(base, _B_PER_W)])


@functools.lru_cache(maxsize=1)
def _make_gather():
    return pl.kernel(
        _gather_body,
        mesh=plsc.VectorSubcoreMesh(core_axis_name="c", subcore_axis_name="s"),
        out_type=[
            jax.ShapeDtypeStruct((BATCH, EMBED), jnp.float32),
            jax.ShapeDtypeStruct((BATCH, EMBED), jnp.float32),
        ],
        scratch_types=[
            pltpu.VMEM((_NCHUNK, _CHUNK), jnp.int32),
            pltpu.VMEM((_NCHUNK, _CHUNK), jnp.int32),
            pltpu.VMEM((_B_PER_W, EMBED), jnp.float32),
            pltpu.VMEM((_B_PER_W, EMBED), jnp.float32),
            pltpu.SemaphoreType.DMA,
            pltpu.SemaphoreType.DMA,
        ],
        compiler_params=pltpu.CompilerParams(use_tc_tiling_on_sc=False),
    )

_BS = 2048


def _mlp_body(ue_ref, be_ref, w1u_ref, w1b_ref, b1_ref, w2_ref, b2_ref, out_ref):
    h = (jnp.dot(ue_ref[:], w1u_ref[:], preferred_element_type=jnp.float32)
         + jnp.dot(be_ref[:], w1b_ref[:], preferred_element_type=jnp.float32)
         + b1_ref[:])
    h = jnp.where(h >= 0, h, 0.01 * h)
    raw = jnp.sum(h * w2_ref[:], axis=1, keepdims=True) + b2_ref[0, 0]
    out_ref[:] = 1.0 + 4.0 * jax.nn.sigmoid(raw)


_mlp = pl.pallas_call(
    _mlp_body,
    grid=(BATCH // _BS,),
    in_specs=[
        pl.BlockSpec((_BS, EMBED), lambda i: (i, 0)),
        pl.BlockSpec((_BS, EMBED), lambda i: (i, 0)),
        pl.BlockSpec((EMBED, HIDDEN), lambda i: (0, 0)),
        pl.BlockSpec((EMBED, HIDDEN), lambda i: (0, 0)),
        pl.BlockSpec((1, HIDDEN), lambda i: (0, 0)),
        pl.BlockSpec((1, HIDDEN), lambda i: (0, 0)),
        pl.BlockSpec(memory_space=pltpu.SMEM),
    ],
    out_specs=pl.BlockSpec((_BS, 1), lambda i: (i, 0)),
    out_shape=jax.ShapeDtypeStruct((BATCH, 1), jnp.float32),
)


def kernel(user_id, book_id, user_emb, book_emb, W1, b1, W2, b2):
    uid = user_id.astype(jnp.int32).reshape(BATCH // _IDX_COLS, _IDX_COLS)
    bid = book_id.astype(jnp.int32).reshape(BATCH // _IDX_COLS, _IDX_COLS)
    ue_g, be_g = _make_gather()(uid, bid, user_emb, book_emb)
    w1u = W1[:, :EMBED].T
    w1b = W1[:, EMBED:].T
    return _mlp(ue_g, be_g, w1u, w1b,
                b1.reshape(1, HIDDEN), W2, b2.reshape(1, 1))
